# baseline (speedup 1.0000x reference)
import functools
import jax
import jax.numpy as jnp
from jax import lax
from jax.experimental import pallas as pl
from jax.experimental.pallas import tpu as pltpu
from jax.experimental.pallas import tpu_sc as plsc

B, H, W, C = 64, 32, 32, 768
N = B * H * W
NW = 32
R_SC = N
N_TC = N - R_SC
ROWS_PW = R_SC // NW
CH = 64
NCHUNK = ROWS_PW // CH
K = C // 16


def _sc_cam_body(x_hbm, w_hbm, p_hbm, wv, xb, ob, sem, osem):
    wid = lax.axis_index("s") * 2 + lax.axis_index("c")
    base = N_TC + wid * ROWS_PW
    pltpu.sync_copy(w_hbm, wv)

    def chunk_src(g):
        return x_hbm.at[pl.ds((base + g * CH) * C, CH * C)]

    def chunk_dst(g):
        return p_hbm.at[pl.ds((wid * ROWS_PW + g * CH) * 16, CH * 16)]

    pltpu.make_async_copy(chunk_src(0), xb.at[0], sem.at[0]).start()

    def chunk_body(g, carry):
        slot = lax.rem(g, 2)
        nslot = lax.rem(g + 1, 2)
        pltpu.make_async_copy(chunk_src(g), xb.at[slot], sem.at[slot]).wait()

        @pl.when(g + 1 < NCHUNK)
        def _():
            pltpu.make_async_copy(chunk_src(g + 1), xb.at[nslot], sem.at[nslot]).start()

        @pl.when(g >= 2)
        def _():
            pltpu.make_async_copy(chunk_dst(g - 2), ob.at[slot], osem.at[slot]).wait()

        for r in range(CH):
            off = r * C
            acc = xb[slot, pl.ds(off, 16)] * wv[pl.ds(0, 16)]
            for k in range(1, K):
                acc = acc + xb[slot, pl.ds(off + k * 16, 16)] * wv[pl.ds(k * 16, 16)]
            ob[slot, pl.ds(r * 16, 16)] = acc

        pltpu.make_async_copy(ob.at[slot], chunk_dst(g), osem.at[slot]).start()
        return carry

    lax.fori_loop(0, NCHUNK, chunk_body, 0)
    pltpu.make_async_copy(chunk_dst(NCHUNK - 2), ob.at[lax.rem(NCHUNK - 2, 2)], osem.at[lax.rem(NCHUNK - 2, 2)]).wait()
    pltpu.make_async_copy(chunk_dst(NCHUNK - 1), ob.at[lax.rem(NCHUNK - 1, 2)], osem.at[lax.rem(NCHUNK - 1, 2)]).wait()


_sc_cam = functools.partial(
    pl.kernel,
    out_type=jax.ShapeDtypeStruct((R_SC * 16,), jnp.float32),
    mesh=plsc.VectorSubcoreMesh(core_axis_name="c", subcore_axis_name="s"),
    scratch_types=[
        pltpu.VMEM((C,), jnp.float32),
        pltpu.VMEM((2, CH * C), jnp.float32),
        pltpu.VMEM((2, CH * 16), jnp.float32),
        pltpu.SemaphoreType.DMA((2,)),
        pltpu.SemaphoreType.DMA((2,)),
    ],
)(_sc_cam_body)


def _reduce_body(p_ref, o_ref):
    r = jnp.sum(p_ref[...], axis=1)
    o_ref[...] = r.reshape(r.shape[0] // 128, 128)


def kernel(conv_input, output, weight):
    x = conv_input.reshape(N * C)
    parts = _sc_cam(x, weight)
    RED_ROWS = 16384
    out = pl.pallas_call(
        _reduce_body,
        grid=(R_SC // RED_ROWS,),
        in_specs=[pl.BlockSpec((RED_ROWS, 16), lambda i: (i, 0))],
        out_specs=pl.BlockSpec((RED_ROWS // 128, 128), lambda i: (i, 0)),
        out_shape=jax.ShapeDtypeStruct((R_SC // 128, 128), jnp.float32),
    )(parts.reshape(R_SC, 16))
    return (out.reshape(B, H, W), output)


# speedup vs baseline: 2.6068x; 2.6068x over previous
import functools
import jax
import jax.numpy as jnp
from jax import lax
from jax.experimental import pallas as pl
from jax.experimental.pallas import tpu as pltpu
from jax.experimental.pallas import tpu_sc as plsc

B, H, W, C = 64, 32, 32, 768
N = B * H * W
LANES = 128

R_SC = 4096
N_TC = N - R_SC

ROWS = 2048


def _tc_body(x_ref, w_ref, o_ref):
    r = jnp.sum(x_ref[...] * w_ref[...], axis=1)
    o_ref[...] = r.reshape(ROWS // LANES, LANES)


NW = 32
ROWS_PW = R_SC // NW
CH = 32
NCHUNK = ROWS_PW // CH
K = C // 16


def _sc_body(x_hbm, w_hbm, p_hbm, wv, xb, ob, sem, osem):
    wid = lax.axis_index("s") * 2 + lax.axis_index("c")
    base = N_TC + wid * ROWS_PW
    pltpu.sync_copy(w_hbm, wv)

    def chunk_src(g):
        return x_hbm.at[pl.ds((base + g * CH) * C, CH * C)]

    def chunk_dst(g):
        return p_hbm.at[pl.ds((wid * ROWS_PW + g * CH) * 16, CH * 16)]

    pltpu.make_async_copy(chunk_src(0), xb.at[0], sem.at[0]).start()

    def chunk_body(g, carry):
        slot = lax.rem(g, 2)
        nslot = lax.rem(g + 1, 2)
        pltpu.make_async_copy(chunk_src(g), xb.at[slot], sem.at[slot]).wait()

        @pl.when(g + 1 < NCHUNK)
        def _():
            pltpu.make_async_copy(chunk_src(g + 1), xb.at[nslot], sem.at[nslot]).start()

        @pl.when(g >= 2)
        def _():
            pltpu.make_async_copy(chunk_dst(g - 2), ob.at[slot], osem.at[slot]).wait()

        for r in range(CH):
            off = r * C
            acc = xb[slot, pl.ds(off, 16)] * wv[pl.ds(0, 16)]
            for k in range(1, K):
                acc = acc + xb[slot, pl.ds(off + k * 16, 16)] * wv[pl.ds(k * 16, 16)]
            ob[slot, pl.ds(r * 16, 16)] = acc

        pltpu.make_async_copy(ob.at[slot], chunk_dst(g), osem.at[slot]).start()
        return carry

    lax.fori_loop(0, NCHUNK, chunk_body, 0)
    pltpu.make_async_copy(chunk_dst(NCHUNK - 2), ob.at[lax.rem(NCHUNK - 2, 2)],
                          osem.at[lax.rem(NCHUNK - 2, 2)]).wait()
    pltpu.make_async_copy(chunk_dst(NCHUNK - 1), ob.at[lax.rem(NCHUNK - 1, 2)],
                          osem.at[lax.rem(NCHUNK - 1, 2)]).wait()


_sc_cam = functools.partial(
    pl.kernel,
    out_type=jax.ShapeDtypeStruct((R_SC * 16,), jnp.float32),
    mesh=plsc.VectorSubcoreMesh(core_axis_name="c", subcore_axis_name="s"),
    scratch_types=[
        pltpu.VMEM((C,), jnp.float32),
        pltpu.VMEM((2, CH * C), jnp.float32),
        pltpu.VMEM((2, CH * 16), jnp.float32),
        pltpu.SemaphoreType.DMA((2,)),
        pltpu.SemaphoreType.DMA((2,)),
    ],
)(_sc_body)


def _reduce_body(p_ref, o_ref):
    r = jnp.sum(p_ref[...], axis=1)
    o_ref[...] = r.reshape(r.shape[0] // 128, 128)


def kernel(conv_input, output, weight):
    xf = conv_input.reshape(N * C)
    x = conv_input.reshape(N, C)
    w = weight.reshape(1, C)

    parts = _sc_cam(xf, weight)

    out_tc = pl.pallas_call(
        _tc_body,
        grid=(N_TC // ROWS,),
        in_specs=[
            pl.BlockSpec((ROWS, C), lambda i: (i, 0)),
            pl.BlockSpec((1, C), lambda i: (0, 0)),
        ],
        out_specs=pl.BlockSpec((ROWS // LANES, LANES), lambda i: (i, 0)),
        out_shape=jax.ShapeDtypeStruct((N_TC // LANES, LANES), jnp.float32),
    )(x, w)

    out_sc = pl.pallas_call(
        _reduce_body,
        grid=(1,),
        in_specs=[pl.BlockSpec((R_SC, 16), lambda i: (0, 0))],
        out_specs=pl.BlockSpec((R_SC // LANES, LANES), lambda i: (0, 0)),
        out_shape=jax.ShapeDtypeStruct((R_SC // LANES, LANES), jnp.float32),
    )(parts.reshape(R_SC, 16))

    out = jnp.concatenate([out_tc, out_sc], axis=0)
    return (out.reshape(B, H, W), output)


# speedup vs baseline: 10.9603x; 4.2046x over previous
import jax
import jax.numpy as jnp
from jax.experimental import pallas as pl
from jax.experimental.pallas import tpu as pltpu

B, H, W, C = 64, 32, 32, 768
N = B * H * W
LANES = 128
ROWS = 1024
GRID = N // ROWS
NBUF = 4


def _cam_body(x_hbm, w_ref, o_ref, buf, sem):
    i = pl.program_id(0)
    slot = jax.lax.rem(i, NBUF)

    @pl.when(i == 0)
    def _prime():
        for j in range(NBUF):
            pltpu.make_async_copy(x_hbm.at[j], buf.at[j], sem.at[j]).start()

    pltpu.make_async_copy(x_hbm.at[i], buf.at[slot], sem.at[slot]).wait()
    r = jnp.sum(buf[slot] * w_ref[...], axis=1)
    o_ref[...] = r.reshape(ROWS // LANES, LANES)

    @pl.when(i + NBUF < GRID)
    def _refill():
        pltpu.make_async_copy(x_hbm.at[i + NBUF], buf.at[slot], sem.at[slot]).start()


def kernel(conv_input, output, weight):
    x = conv_input.reshape(GRID, ROWS, C)
    w = weight.reshape(1, C)
    out = pl.pallas_call(
        _cam_body,
        grid=(GRID,),
        in_specs=[
            pl.BlockSpec(memory_space=pl.ANY),
            pl.BlockSpec((1, C), lambda i: (0, 0)),
        ],
        out_specs=pl.BlockSpec((ROWS // LANES, LANES), lambda i: (i, 0)),
        out_shape=jax.ShapeDtypeStruct((N // LANES, LANES), jnp.float32),
        scratch_shapes=[
            pltpu.VMEM((NBUF, ROWS, C), jnp.float32),
            pltpu.SemaphoreType.DMA((NBUF,)),
        ],
    )(x, w)
    return (out.reshape(B, H, W), output)
